# baseline (speedup 1.0000x reference)
import functools

import jax
import jax.numpy as jnp
from jax import lax
from jax.experimental import pallas as pl
from jax.experimental.pallas import tpu as pltpu
from jax.experimental.pallas import tpu_sc as plsc

D = 128
DE = 16
NUM_STEPS = 8

NC = 2
NS = 16
NW = NC * NS

CHUNK = 40


def _mm_body(x_ref, w_ref, b_ref, o_ref):
  acc = (
      jnp.dot(x_ref[...], w_ref[...], preferred_element_type=jnp.float32)
      + b_ref[...]
  )
  o_ref[...] = acc.astype(o_ref.dtype)


def _matmul_bias(x, w, b, block_rows, out_dtype=jnp.float32):
  m, k = x.shape
  n = w.shape[1]
  grid = m // block_rows
  return pl.pallas_call(
      _mm_body,
      grid=(grid,),
      in_specs=[
          pl.BlockSpec((block_rows, k), lambda i: (i, 0)),
          pl.BlockSpec((k, n), lambda i: (0, 0)),
          pl.BlockSpec((1, n), lambda i: (0, 0)),
      ],
      out_specs=pl.BlockSpec((block_rows, n), lambda i: (i, 0)),
      out_shape=jax.ShapeDtypeStruct((m, n), out_dtype),
  )(x, w, b.reshape(1, n))


def _gru_body(p0_ref, p1_ref, h_ref, gh_ref, wi_ref, bi_ref, w1_ref, wh_ref,
              bh_ref, h2_ref, hw2_ref, gh2_ref):
  agg = p0_ref[...] + p1_ref[...]
  gx = jnp.dot(agg, wi_ref[...], preferred_element_type=jnp.float32) + bi_ref[...]
  gh = gh_ref[...]
  h = h_ref[...]
  xr, xz, xn = gx[:, :D], gx[:, D:2 * D], gx[:, 2 * D:]
  hr, hz, hn = gh[:, :D], gh[:, D:2 * D], gh[:, 2 * D:]
  r = jax.nn.sigmoid(xr + hr)
  z = jax.nn.sigmoid(xz + hz)
  n = jnp.tanh(xn + r * hn)
  h2 = (1.0 - z) * n + z * h
  h2_ref[...] = h2
  hw2_ref[...] = jnp.dot(h2, w1_ref[...], preferred_element_type=jnp.float32)
  gh2_ref[...] = (
      jnp.dot(h2, wh_ref[...], preferred_element_type=jnp.float32) + bh_ref[...]
  )


def _gru_update(p0, p1, h, gh, W_i, b_i, W1, W_h, b_h, block_rows):
  n_nodes = h.shape[0]
  grid = n_nodes // block_rows
  return pl.pallas_call(
      _gru_body,
      grid=(grid,),
      in_specs=[
          pl.BlockSpec((block_rows, D), lambda i: (i, 0)),
          pl.BlockSpec((block_rows, D), lambda i: (i, 0)),
          pl.BlockSpec((block_rows, D), lambda i: (i, 0)),
          pl.BlockSpec((block_rows, 3 * D), lambda i: (i, 0)),
          pl.BlockSpec((D, 3 * D), lambda i: (0, 0)),
          pl.BlockSpec((1, 3 * D), lambda i: (0, 0)),
          pl.BlockSpec((D, D), lambda i: (0, 0)),
          pl.BlockSpec((D, 3 * D), lambda i: (0, 0)),
          pl.BlockSpec((1, 3 * D), lambda i: (0, 0)),
      ],
      out_specs=[
          pl.BlockSpec((block_rows, D), lambda i: (i, 0)),
          pl.BlockSpec((block_rows, D), lambda i: (i, 0)),
          pl.BlockSpec((block_rows, 3 * D), lambda i: (i, 0)),
      ],
      out_shape=[
          jax.ShapeDtypeStruct((n_nodes, D), jnp.float32),
          jax.ShapeDtypeStruct((n_nodes, D), jnp.float32),
          jax.ShapeDtypeStruct((n_nodes, 3 * D), jnp.float32),
      ],
  )(p0, p1, h, gh, W_i, b_i.reshape(1, 3 * D), W1, W_h, b_h.reshape(1, 3 * D))


def _make_edge_call(n_pad, n_edges):
  epw = n_edges // NW
  nch = epw // CHUNK
  rpw = n_pad // NS
  mesh = plsc.VectorSubcoreMesh(
      core_axis_name="c", subcore_axis_name="s", num_cores=NC, num_subcores=NS)

  @functools.partial(
      pl.kernel,
      mesh=mesh,
      out_type=jax.ShapeDtypeStruct((NC * n_pad, D), jnp.float32),
      scratch_types=[
          pltpu.VMEM((CHUNK,), jnp.int32),
          pltpu.VMEM((CHUNK,), jnp.int32),
          pltpu.VMEM((CHUNK,), jnp.int32),
          pltpu.VMEM((CHUNK,), jnp.int32),
          pltpu.VMEM((CHUNK, D), jnp.float32),
          pltpu.VMEM((CHUNK, D), jnp.float32),
          pltpu.VMEM((CHUNK, D), jnp.float32),
          pltpu.VMEM((CHUNK, D), jnp.float32),
          pltpu.VMEM_SHARED((n_pad, D), jnp.float32),
          pltpu.SemaphoreType.DMA,
          pltpu.SemaphoreType.DMA,
          pltpu.SemaphoreType.DMA,
          pltpu.SemaphoreType.DMA,
          pltpu.SemaphoreType.DMA,
          pltpu.SemaphoreType.DMA,
          pltpu.SemaphoreType.DMA,
          pltpu.SemaphoreType.DMA,
          pltpu.SemaphoreType.DMA,
          pltpu.SemaphoreType.DMA,
      ],
  )
  def edge_call(hw_hbm, ebias_hbm, send_hbm, recv_hbm, zeros_hbm, out_hbm,
                sidx0, sidx1, ridx0, ridx1, rows0, rows1, eb0, eb1, agg_sh,
                gsem0, gsem1, esem0, esem1, isem0, isem1, rsem0, rsem1,
                ssem0, ssem1):
    c = lax.axis_index("c")
    s = lax.axis_index("s")
    wid = s * NC + c
    rows = (rows0, rows1)
    ebs = (eb0, eb1)
    sidx = (sidx0, sidx1)
    ridx = (ridx0, ridx1)
    gsems = (gsem0, gsem1)
    esems = (esem0, esem1)
    isems = (isem0, isem1)
    rsems = (rsem0, rsem1)
    ssems = (ssem0, ssem1)

    pltpu.sync_copy(zeros_hbm.at[pl.ds(s * rpw, rpw)],
                    agg_sh.at[pl.ds(s * rpw, rpw)])
    plsc.subcore_barrier()

    base0 = wid * epw

    def eslice(k):
      return pl.ds(base0 + k * CHUNK, CHUNK)

    def issue_sidx(k, buf):
      pltpu.async_copy(send_hbm.at[eslice(k)], sidx[buf], isems[buf])

    def wait_sidx(k, buf):
      pltpu.make_async_copy(send_hbm.at[eslice(k)], sidx[buf],
                            isems[buf]).wait()

    def issue_ridx(k, buf):
      pltpu.async_copy(recv_hbm.at[eslice(k)], ridx[buf], rsems[buf])

    def wait_ridx(k, buf):
      pltpu.make_async_copy(recv_hbm.at[eslice(k)], ridx[buf],
                            rsems[buf]).wait()

    def issue_data(k, buf):
      pltpu.async_copy(hw_hbm.at[sidx[buf]], rows[buf], gsems[buf])
      pltpu.async_copy(ebias_hbm.at[eslice(k)], ebs[buf], esems[buf])

    def wait_scatter(buf):
      pltpu.make_async_copy(rows[buf], agg_sh.at[ridx[buf]],
                            ssems[buf]).wait()

    def process(k, buf):
      pltpu.make_async_copy(hw_hbm.at[sidx[buf]], rows[buf],
                            gsems[buf]).wait()
      pltpu.make_async_copy(ebias_hbm.at[eslice(k)], ebs[buf],
                            esems[buf]).wait()
      wait_ridx(k, buf)

      def e_body(e, acc):
        for j in range(D // 16):
          sl = pl.ds(j * 16, 16)
          v = rows[buf][e, sl] + ebs[buf][e, sl]
          rows[buf][e, sl] = jnp.maximum(v, 0.0)
        return acc

      lax.fori_loop(0, CHUNK, e_body, 0, unroll=2)
      pltpu.async_copy(rows[buf], agg_sh.at[ridx[buf]], ssems[buf], add=True)

    def body(k, buf):
      nxt = 1 - buf
      wait_sidx(k + 1, nxt)
      wait_scatter(nxt)
      issue_ridx(k + 1, nxt)
      issue_data(k + 1, nxt)
      process(k, buf)
      issue_sidx(k + 2, buf)

    issue_sidx(0, 0)
    issue_ridx(0, 0)
    wait_sidx(0, 0)
    issue_data(0, 0)
    issue_sidx(1, 1)
    issue_ridx(1, 1)
    wait_sidx(1, 1)
    issue_data(1, 1)
    process(0, 0)
    issue_sidx(2, 0)
    wait_sidx(2, 0)
    wait_scatter(0)
    issue_ridx(2, 0)
    issue_data(2, 0)
    process(1, 1)
    issue_sidx(3, 1)

    def chunk_pair(k2, carry):
      k = 2 * k2 + 2
      body(k, 0)
      body(k + 1, 1)
      return carry

    lax.fori_loop(0, (nch - 4) // 2, chunk_pair, 0)
    wait_sidx(nch - 1, 1)
    wait_scatter(1)
    issue_ridx(nch - 1, 1)
    issue_data(nch - 1, 1)
    process(nch - 2, 0)
    wait_scatter(0)
    process(nch - 1, 1)
    wait_scatter(1)

    plsc.subcore_barrier()
    pltpu.sync_copy(agg_sh.at[pl.ds(s * rpw, rpw)],
                    out_hbm.at[pl.ds(c * n_pad + s * rpw, rpw)])

  return edge_call


def kernel(nodes, edge_attr, senders, receivers, W_msg, b_msg, W_i, W_h, b_i,
           b_h):
  n_nodes = nodes.shape[0]
  n_edges = senders.shape[0]
  senders = senders.astype(jnp.int32)
  receivers = receivers.astype(jnp.int32)

  W1 = W_msg[:D]
  W2 = W_msg[D:]

  ebias = _matmul_bias(edge_attr, W2, b_msg, block_rows=4000)

  n_pad = ((n_nodes + 1279) // 1280) * 1280
  nodes_p = jnp.pad(nodes, ((0, n_pad - n_nodes), (0, 0)))

  Wcat = jnp.concatenate([W1, W_h], axis=1)
  bcat = jnp.concatenate([jnp.zeros((D,), jnp.float32), b_h])
  hwgh = _matmul_bias(nodes_p, Wcat, bcat, block_rows=1280)
  hW0, gh0 = hwgh[:, :D], hwgh[:, D:]

  zeros = jnp.zeros((n_pad, D), jnp.float32)
  edge_call = _make_edge_call(n_pad, n_edges)

  def step(carry, _):
    h, hW, gh = carry
    p = edge_call(hW, ebias, senders, receivers, zeros)
    h2, hW2, gh2 = _gru_update(p[:n_pad], p[n_pad:], h, gh, W_i, b_i, W1,
                               W_h, b_h, block_rows=1280)
    return (h2, hW2, gh2), None

  (h_final, _, _), _ = lax.scan(step, (nodes_p, hW0, gh0),
                                jnp.arange(NUM_STEPS))
  return h_final[:n_nodes]


# speedup vs baseline: 1.9261x; 1.9261x over previous
import functools

import jax
import jax.numpy as jnp
from jax import lax
from jax.experimental import pallas as pl
from jax.experimental.pallas import tpu as pltpu
from jax.experimental.pallas import tpu_sc as plsc

D = 128
DE = 16
NUM_STEPS = 8

NC = 2
NS = 16
NW = NC * NS

CHUNK = 40


def _mm_body(x_ref, w_ref, b_ref, o_ref):
  acc = (
      jnp.dot(x_ref[...], w_ref[...], preferred_element_type=jnp.float32)
      + b_ref[...]
  )
  o_ref[...] = acc.astype(o_ref.dtype)


def _matmul_bias(x, w, b, block_rows, out_dtype=jnp.float32):
  m, k = x.shape
  n = w.shape[1]
  grid = m // block_rows
  return pl.pallas_call(
      _mm_body,
      grid=(grid,),
      in_specs=[
          pl.BlockSpec((block_rows, k), lambda i: (i, 0)),
          pl.BlockSpec((k, n), lambda i: (0, 0)),
          pl.BlockSpec((1, n), lambda i: (0, 0)),
      ],
      out_specs=pl.BlockSpec((block_rows, n), lambda i: (i, 0)),
      out_shape=jax.ShapeDtypeStruct((m, n), out_dtype),
  )(x, w, b.reshape(1, n))


def _gru_body(p0_ref, p1_ref, h_ref, gh_ref, wi_ref, bi_ref, w1_ref, wh_ref,
              bh_ref, h2_ref, hw2_ref, gh2_ref):
  agg = p0_ref[...] + p1_ref[...]
  gx = jnp.dot(agg, wi_ref[...], preferred_element_type=jnp.float32) + bi_ref[...]
  gh = gh_ref[...]
  h = h_ref[...]
  xr, xz, xn = gx[:, :D], gx[:, D:2 * D], gx[:, 2 * D:]
  hr, hz, hn = gh[:, :D], gh[:, D:2 * D], gh[:, 2 * D:]
  r = jax.nn.sigmoid(xr + hr)
  z = jax.nn.sigmoid(xz + hz)
  n = jnp.tanh(xn + r * hn)
  h2 = (1.0 - z) * n + z * h
  h2_ref[...] = h2
  hw2_ref[...] = jnp.dot(h2, w1_ref[...], preferred_element_type=jnp.float32)
  gh2_ref[...] = (
      jnp.dot(h2, wh_ref[...], preferred_element_type=jnp.float32) + bh_ref[...]
  )


def _gru_update(p0, p1, h, gh, W_i, b_i, W1, W_h, b_h, block_rows):
  n_nodes = h.shape[0]
  grid = n_nodes // block_rows
  return pl.pallas_call(
      _gru_body,
      grid=(grid,),
      in_specs=[
          pl.BlockSpec((block_rows, D), lambda i: (i, 0)),
          pl.BlockSpec((block_rows, D), lambda i: (i, 0)),
          pl.BlockSpec((block_rows, D), lambda i: (i, 0)),
          pl.BlockSpec((block_rows, 3 * D), lambda i: (i, 0)),
          pl.BlockSpec((D, 3 * D), lambda i: (0, 0)),
          pl.BlockSpec((1, 3 * D), lambda i: (0, 0)),
          pl.BlockSpec((D, D), lambda i: (0, 0)),
          pl.BlockSpec((D, 3 * D), lambda i: (0, 0)),
          pl.BlockSpec((1, 3 * D), lambda i: (0, 0)),
      ],
      out_specs=[
          pl.BlockSpec((block_rows, D), lambda i: (i, 0)),
          pl.BlockSpec((block_rows, D), lambda i: (i, 0)),
          pl.BlockSpec((block_rows, 3 * D), lambda i: (i, 0)),
      ],
      out_shape=[
          jax.ShapeDtypeStruct((n_nodes, D), jnp.float32),
          jax.ShapeDtypeStruct((n_nodes, D), jnp.float32),
          jax.ShapeDtypeStruct((n_nodes, 3 * D), jnp.float32),
      ],
  )(p0, p1, h, gh, W_i, b_i.reshape(1, 3 * D), W1, W_h, b_h.reshape(1, 3 * D))


def _make_edge_call(n_pad, n_edges):
  epw = n_edges // NW
  nch = epw // CHUNK
  rpw = n_pad // NS
  mesh = plsc.VectorSubcoreMesh(
      core_axis_name="c", subcore_axis_name="s", num_cores=NC, num_subcores=NS)

  @functools.partial(
      pl.kernel,
      mesh=mesh,
      out_type=jax.ShapeDtypeStruct((NC * n_pad, D), jnp.float32),
      scratch_types=[
          pltpu.VMEM((CHUNK,), jnp.int32),
          pltpu.VMEM((CHUNK,), jnp.int32),
          pltpu.VMEM((CHUNK,), jnp.int32),
          pltpu.VMEM((CHUNK,), jnp.int32),
          pltpu.VMEM((CHUNK, D), jnp.float32),
          pltpu.VMEM((CHUNK, D), jnp.float32),
          pltpu.VMEM((CHUNK, D), jnp.float32),
          pltpu.VMEM((CHUNK, D), jnp.float32),
          pltpu.VMEM_SHARED((n_pad, D), jnp.float32),
          pltpu.SemaphoreType.DMA,
          pltpu.SemaphoreType.DMA,
          pltpu.SemaphoreType.DMA,
          pltpu.SemaphoreType.DMA,
          pltpu.SemaphoreType.DMA,
          pltpu.SemaphoreType.DMA,
          pltpu.SemaphoreType.DMA,
          pltpu.SemaphoreType.DMA,
          pltpu.SemaphoreType.DMA,
          pltpu.SemaphoreType.DMA,
      ],
  )
  def edge_call(hw_hbm, ebias_hbm, send_hbm, recv_hbm, zeros_hbm, out_hbm,
                sidx0, sidx1, ridx0, ridx1, rows0, rows1, eb0, eb1, agg_sh,
                gsem0, gsem1, esem0, esem1, isem0, isem1, rsem0, rsem1,
                ssem0, ssem1):
    c = lax.axis_index("c")
    s = lax.axis_index("s")
    wid = s * NC + c
    rows = (rows0, rows1)
    ebs = (eb0, eb1)
    sidx = (sidx0, sidx1)
    ridx = (ridx0, ridx1)
    gsems = (gsem0, gsem1)
    esems = (esem0, esem1)
    isems = (isem0, isem1)
    rsems = (rsem0, rsem1)
    ssems = (ssem0, ssem1)

    pltpu.sync_copy(zeros_hbm.at[pl.ds(s * rpw, rpw)],
                    agg_sh.at[pl.ds(s * rpw, rpw)])
    plsc.subcore_barrier()

    base0 = wid * epw

    def eslice(k):
      return pl.ds(base0 + k * CHUNK, CHUNK)

    def issue_sidx(k, buf):
      pltpu.async_copy(send_hbm.at[eslice(k)], sidx[buf], isems[buf])

    def wait_sidx(k, buf):
      pltpu.make_async_copy(send_hbm.at[eslice(k)], sidx[buf],
                            isems[buf]).wait()

    def issue_ridx(k, buf):
      pltpu.async_copy(recv_hbm.at[eslice(k)], ridx[buf], rsems[buf])

    def wait_ridx(k, buf):
      pltpu.make_async_copy(recv_hbm.at[eslice(k)], ridx[buf],
                            rsems[buf]).wait()

    def issue_data(k, buf):
      pltpu.async_copy(hw_hbm.at[sidx[buf]], rows[buf], gsems[buf])
      pltpu.async_copy(ebias_hbm.at[eslice(k)], ebs[buf], esems[buf])

    def wait_scatter(buf):
      pltpu.make_async_copy(rows[buf], agg_sh.at[ridx[buf]],
                            ssems[buf]).wait()

    def process(k, buf):
      pltpu.make_async_copy(hw_hbm.at[sidx[buf]], rows[buf],
                            gsems[buf]).wait()
      pltpu.make_async_copy(ebias_hbm.at[eslice(k)], ebs[buf],
                            esems[buf]).wait()
      wait_ridx(k, buf)

      def e_body(e, acc):
        for j in range(D // 16):
          sl = pl.ds(j * 16, 16)
          v = rows[buf][e, sl] + ebs[buf][e, sl]
          rows[buf][e, sl] = jnp.maximum(v, 0.0)
        return acc

      lax.fori_loop(0, CHUNK, e_body, 0)
      pltpu.async_copy(rows[buf], agg_sh.at[ridx[buf]], ssems[buf], add=True)

    def body(k, buf):
      nxt = 1 - buf
      wait_sidx(k + 1, nxt)
      wait_scatter(nxt)
      issue_ridx(k + 1, nxt)
      issue_data(k + 1, nxt)
      process(k, buf)
      issue_sidx(k + 2, buf)

    issue_sidx(0, 0)
    issue_ridx(0, 0)
    wait_sidx(0, 0)
    issue_data(0, 0)
    issue_sidx(1, 1)
    issue_ridx(1, 1)
    wait_sidx(1, 1)
    issue_data(1, 1)
    process(0, 0)
    issue_sidx(2, 0)
    wait_sidx(2, 0)
    wait_scatter(0)
    issue_ridx(2, 0)
    issue_data(2, 0)
    process(1, 1)
    issue_sidx(3, 1)

    def chunk_pair(k2, carry):
      k = 2 * k2 + 2
      body(k, 0)
      body(k + 1, 1)
      return carry

    lax.fori_loop(0, (nch - 4) // 2, chunk_pair, 0)
    wait_sidx(nch - 1, 1)
    wait_scatter(1)
    issue_ridx(nch - 1, 1)
    issue_data(nch - 1, 1)
    process(nch - 2, 0)
    wait_scatter(0)
    process(nch - 1, 1)
    wait_scatter(1)

    plsc.subcore_barrier()
    pltpu.sync_copy(agg_sh.at[pl.ds(s * rpw, rpw)],
                    out_hbm.at[pl.ds(c * n_pad + s * rpw, rpw)])

  return edge_call


def kernel(nodes, edge_attr, senders, receivers, W_msg, b_msg, W_i, W_h, b_i,
           b_h):
  n_nodes = nodes.shape[0]
  n_edges = senders.shape[0]
  senders = senders.astype(jnp.int32)
  receivers = receivers.astype(jnp.int32)

  W1 = W_msg[:D]
  W2 = W_msg[D:]

  ebias = _matmul_bias(edge_attr, W2, b_msg, block_rows=4000)

  n_pad = ((n_nodes + 1279) // 1280) * 1280
  nodes_p = jnp.pad(nodes, ((0, n_pad - n_nodes), (0, 0)))

  Wcat = jnp.concatenate([W1, W_h], axis=1)
  bcat = jnp.concatenate([jnp.zeros((D,), jnp.float32), b_h])
  hwgh = _matmul_bias(nodes_p, Wcat, bcat, block_rows=1280)
  hW0, gh0 = hwgh[:, :D], hwgh[:, D:]

  zeros = jnp.zeros((n_pad, D), jnp.float32)
  edge_call = _make_edge_call(n_pad, n_edges)

  def step(carry, _):
    h, hW, gh = carry
    p = edge_call(hW, ebias, senders, receivers, zeros)
    h2, hW2, gh2 = _gru_update(p[:n_pad], p[n_pad:], h, gh, W_i, b_i, W1,
                               W_h, b_h, block_rows=1280)
    return (h2, hW2, gh2), None

  (h_final, _, _), _ = lax.scan(step, (nodes_p, hW0, gh0),
                                jnp.arange(NUM_STEPS))
  return h_final[:n_nodes]
